# baseline (speedup 1.0000x reference)
import functools

import jax
import jax.numpy as jnp
from jax import lax
from jax.experimental import pallas as pl
from jax.experimental.pallas import tpu as pltpu
from jax.experimental.pallas import tpu_sc as plsc

NC = 2
NS = 16
NW = NC * NS
R = 12
F = 8
TBLK = 16384


def _tr_body(in_ref, o_ref):
    t = in_ref[...].T
    o_ref[...] = jnp.concatenate([t, t], axis=1)


def _transpose_table(WT):
    D, V = WT.shape
    grid = (V + TBLK - 1) // TBLK
    return pl.pallas_call(
        _tr_body,
        out_shape=jax.ShapeDtypeStruct((V, 2 * D), jnp.float32),
        grid=(grid,),
        in_specs=[pl.BlockSpec((D, TBLK), lambda g: (0, g))],
        out_specs=pl.BlockSpec((TBLK, 2 * D), lambda g: (g, 0)),
    )(WT)


def _emb_kernel_factory(B, S, D):
    n_ch = B // NW
    mesh = plsc.VectorSubcoreMesh(core_axis_name="c", subcore_axis_name="s")

    @functools.partial(
        pl.kernel,
        mesh=mesh,
        out_type=jax.ShapeDtypeStruct((B, S, D), jnp.float32),
        scratch_types=[
            pltpu.VMEM((n_ch, S), jnp.int32),
            pltpu.VMEM((R, S, 2 * D), jnp.float32),
            pltpu.SemaphoreType.DMA((R,)),
            pltpu.SemaphoreType.DMA((R,)),
        ],
        compiler_params=pltpu.CompilerParams(use_tc_tiling_on_sc=False),
    )
    def emb(x_hbm, w_hbm, out_hbm, idx_v, buf, gsem, osem):
        wid = lax.axis_index("s") * NC + lax.axis_index("c")
        base = wid * n_ch
        pltpu.sync_copy(x_hbm.at[pl.ds(base, n_ch)], idx_v)

        def fire_gather(j):
            s = lax.rem(j, R)
            pltpu.async_copy(w_hbm.at[idx_v.at[j]], buf.at[s], gsem.at[s])

        def gather_done(j):
            s = lax.rem(j, R)
            pltpu.make_async_copy(w_hbm.at[idx_v.at[j]], buf.at[s],
                                  gsem.at[s]).wait()

        def fire_out(j):
            s = lax.rem(j, R)
            pltpu.async_copy(buf.at[s, :, pl.ds(0, D)],
                             out_hbm.at[base + j], osem.at[s])

        def out_done(j):
            s = lax.rem(j, R)
            pltpu.make_async_copy(buf.at[s, :, pl.ds(0, D)],
                                  out_hbm.at[base + j], osem.at[s]).wait()

        for j in range(F):
            fire_gather(j)

        def body(j, carry):
            gather_done(j)

            @pl.when(j >= 1)
            def _():
                out_done(j - 1)

            @pl.when(j + F < n_ch)
            def _():
                fire_gather(j + F)

            fire_out(j)
            return carry

        lax.fori_loop(0, n_ch, body, 0)
        out_done(n_ch - 1)

    return emb


def kernel(x, W):
    B, S = x.shape
    V, D = W.shape
    assert B % NW == 0
    w_pad = jnp.pad(W, ((0, 0), (0, D)))
    return _emb_kernel_factory(B, S, D)(x.astype(jnp.int32), w_pad)


# speedup vs baseline: 1.5480x; 1.5480x over previous
import functools

import jax
import jax.numpy as jnp
from jax import lax
from jax.experimental import pallas as pl
from jax.experimental.pallas import tpu as pltpu
from jax.experimental.pallas import tpu_sc as plsc

NC = 2
NS = 16
NW = NC * NS
R = 12
F = 8
TBLK = 16384


def _tr_body(in_ref, o_ref):
    t = in_ref[...].T
    o_ref[...] = jnp.concatenate([t, t], axis=1)


def _transpose_table(WT):
    D, V = WT.shape
    grid = (V + TBLK - 1) // TBLK
    return pl.pallas_call(
        _tr_body,
        out_shape=jax.ShapeDtypeStruct((V, 2 * D), jnp.float32),
        grid=(grid,),
        in_specs=[pl.BlockSpec((D, TBLK), lambda g: (0, g))],
        out_specs=pl.BlockSpec((TBLK, 2 * D), lambda g: (g, 0)),
    )(WT)


def _emb_kernel_factory(B, S, D):
    n_ch = B // NW
    mesh = plsc.VectorSubcoreMesh(core_axis_name="c", subcore_axis_name="s")

    @functools.partial(
        pl.kernel,
        mesh=mesh,
        out_type=jax.ShapeDtypeStruct((B, S, D), jnp.float32),
        scratch_types=[
            pltpu.VMEM((n_ch, S), jnp.int32),
            pltpu.VMEM((R, S, D), jnp.float32),
            pltpu.SemaphoreType.DMA((R,)),
            pltpu.SemaphoreType.DMA((R,)),
        ],
        compiler_params=pltpu.CompilerParams(use_tc_tiling_on_sc=False),
    )
    def emb(x_hbm, w_hbm, out_hbm, idx_v, buf, gsem, osem):
        wid = lax.axis_index("s") * NC + lax.axis_index("c")
        base = wid * n_ch
        pltpu.sync_copy(x_hbm.at[pl.ds(base, n_ch)], idx_v)

        def fire_gather(j):
            s = lax.rem(j, R)
            pltpu.async_copy(w_hbm.at[idx_v.at[j]], buf.at[s], gsem.at[s])

        def gather_done(j):
            s = lax.rem(j, R)
            pltpu.make_async_copy(w_hbm.at[idx_v.at[j]], buf.at[s],
                                  gsem.at[s]).wait()

        def fire_out(j):
            s = lax.rem(j, R)
            pltpu.async_copy(buf.at[s], out_hbm.at[base + j], osem.at[s])

        def out_done(j):
            s = lax.rem(j, R)
            pltpu.make_async_copy(buf.at[s], out_hbm.at[base + j],
                                  osem.at[s]).wait()

        for j in range(F):
            fire_gather(j)

        def body(j, carry):
            gather_done(j)

            @pl.when(j >= 1)
            def _():
                out_done(j - 1)

            @pl.when(j + F < n_ch)
            def _():
                fire_gather(j + F)

            fire_out(j)
            return carry

        lax.fori_loop(0, n_ch, body, 0)
        out_done(n_ch - 1)

    return emb


def kernel(x, W):
    B, S = x.shape
    V, D = W.shape
    assert B % NW == 0
    w_half = _transpose_table(W.T).reshape(2 * V, D)
    xi = x.astype(jnp.int32)
    return _emb_kernel_factory(B, S, D)(xi + xi, w_half)


# speedup vs baseline: 1.7052x; 1.1016x over previous
import functools

import jax
import jax.numpy as jnp
from jax import lax
from jax.experimental import pallas as pl
from jax.experimental.pallas import tpu as pltpu
from jax.experimental.pallas import tpu_sc as plsc

NC = 2
NS = 16
NW = NC * NS
R = 12
F = 8
TBLK = 16384


def _tr_body(in_ref, o_ref):
    t = in_ref[...].T
    o_ref[:, 0:t.shape[1]] = t


def _transpose_table(WT):
    D, V = WT.shape
    grid = (V + TBLK - 1) // TBLK
    return pl.pallas_call(
        _tr_body,
        out_shape=jax.ShapeDtypeStruct((V, 2 * D), jnp.float32),
        grid=(grid,),
        in_specs=[pl.BlockSpec((D, TBLK), lambda g: (0, g))],
        out_specs=pl.BlockSpec((TBLK, 2 * D), lambda g: (g, 0)),
    )(WT)


def _emb_kernel_factory(B, S, D):
    n_ch = B // NW
    mesh = plsc.VectorSubcoreMesh(core_axis_name="c", subcore_axis_name="s")

    @functools.partial(
        pl.kernel,
        mesh=mesh,
        out_type=jax.ShapeDtypeStruct((B, S, D), jnp.float32),
        scratch_types=[
            pltpu.VMEM((n_ch, S), jnp.int32),
            pltpu.VMEM((R, S, D), jnp.float32),
            pltpu.SemaphoreType.DMA((R,)),
            pltpu.SemaphoreType.DMA((R,)),
        ],
        compiler_params=pltpu.CompilerParams(use_tc_tiling_on_sc=False),
    )
    def emb(x_hbm, w_hbm, out_hbm, idx_v, buf, gsem, osem):
        wid = lax.axis_index("s") * NC + lax.axis_index("c")
        base = wid * n_ch
        pltpu.sync_copy(x_hbm.at[pl.ds(base, n_ch)], idx_v)

        def fire_gather(j):
            s = lax.rem(j, R)
            pltpu.async_copy(w_hbm.at[idx_v.at[j]], buf.at[s], gsem.at[s])

        def gather_done(j):
            s = lax.rem(j, R)
            pltpu.make_async_copy(w_hbm.at[idx_v.at[j]], buf.at[s],
                                  gsem.at[s]).wait()

        def fire_out(j):
            s = lax.rem(j, R)
            pltpu.async_copy(buf.at[s], out_hbm.at[base + j], osem.at[s])

        def out_done(j):
            s = lax.rem(j, R)
            pltpu.make_async_copy(buf.at[s], out_hbm.at[base + j],
                                  osem.at[s]).wait()

        for j in range(F):
            fire_gather(j)

        def body(j, carry):
            gather_done(j)

            @pl.when(j >= 1)
            def _():
                out_done(j - 1)

            @pl.when(j + F < n_ch)
            def _():
                fire_gather(j + F)

            fire_out(j)
            return carry

        lax.fori_loop(0, n_ch, body, 0)
        out_done(n_ch - 1)

    return emb


def kernel(x, W):
    B, S = x.shape
    V, D = W.shape
    assert B % NW == 0
    w_half = _transpose_table(W.T).reshape(2 * V, D)
    xi = x.astype(jnp.int32)
    return _emb_kernel_factory(B, S, D)(xi + xi, w_half)


# speedup vs baseline: 1.7226x; 1.0102x over previous
import functools

import jax
import jax.numpy as jnp
from jax import lax
from jax.experimental import pallas as pl
from jax.experimental.pallas import tpu as pltpu
from jax.experimental.pallas import tpu_sc as plsc

NC = 2
NS = 16
NW = NC * NS
R = 12
F = 8
TBLK = 24576


def _tr_body(in_ref, o_ref):
    t = in_ref[...].T
    o_ref[:, 0:t.shape[1]] = t


def _transpose_table(WT):
    D, V = WT.shape
    grid = (V + TBLK - 1) // TBLK
    return pl.pallas_call(
        _tr_body,
        out_shape=jax.ShapeDtypeStruct((V, 2 * D), jnp.float32),
        grid=(grid,),
        in_specs=[pl.BlockSpec((D, TBLK), lambda g: (0, g))],
        out_specs=pl.BlockSpec((TBLK, 2 * D), lambda g: (g, 0)),
    )(WT)


def _emb_kernel_factory(B, S, D):
    n_ch = B // NW
    mesh = plsc.VectorSubcoreMesh(core_axis_name="c", subcore_axis_name="s")

    @functools.partial(
        pl.kernel,
        mesh=mesh,
        out_type=jax.ShapeDtypeStruct((B, S, D), jnp.float32),
        scratch_types=[
            pltpu.VMEM((n_ch, S), jnp.int32),
            pltpu.VMEM((R, S, D), jnp.float32),
            pltpu.SemaphoreType.DMA((R,)),
            pltpu.SemaphoreType.DMA((R,)),
        ],
        compiler_params=pltpu.CompilerParams(use_tc_tiling_on_sc=False),
    )
    def emb(x_hbm, w_hbm, out_hbm, idx_v, buf, gsem, osem):
        wid = lax.axis_index("s") * NC + lax.axis_index("c")
        base = wid * n_ch
        pltpu.sync_copy(x_hbm.at[pl.ds(base, n_ch)], idx_v)

        def fire_gather(j):
            s = lax.rem(j, R)
            pltpu.async_copy(w_hbm.at[idx_v.at[j]], buf.at[s], gsem.at[s])

        def gather_done(j):
            s = lax.rem(j, R)
            pltpu.make_async_copy(w_hbm.at[idx_v.at[j]], buf.at[s],
                                  gsem.at[s]).wait()

        def fire_out(j):
            s = lax.rem(j, R)
            pltpu.async_copy(buf.at[s], out_hbm.at[base + j], osem.at[s])

        def out_done(j):
            s = lax.rem(j, R)
            pltpu.make_async_copy(buf.at[s], out_hbm.at[base + j],
                                  osem.at[s]).wait()

        for j in range(F):
            fire_gather(j)

        def body(j, carry):
            gather_done(j)

            @pl.when(j >= 1)
            def _():
                out_done(j - 1)

            @pl.when(j + F < n_ch)
            def _():
                fire_gather(j + F)

            fire_out(j)
            return carry

        lax.fori_loop(0, n_ch, body, 0)
        out_done(n_ch - 1)

    return emb


def kernel(x, W):
    B, S = x.shape
    V, D = W.shape
    assert B % NW == 0
    w_half = _transpose_table(W.T).reshape(2 * V, D)
    xi = x.astype(jnp.int32)
    return _emb_kernel_factory(B, S, D)(xi + xi, w_half)
